# baseline (speedup 1.0000x reference)
import dataclasses
import functools

import jax
import jax.numpy as jnp
import numpy as np
from jax import lax
from jax.experimental import pallas as pl
from jax.experimental.pallas import tpu as pltpu
from jax.experimental.pallas import tpu_sc as plsc

HIDDEN = 256
N_NODES = 10000
N_PAD = 10240
N_EDGES = 160000
SHIFT = float(np.log(2.0))

NC = 2
NS = 16
NW = NC * NS
RPW = HIDDEN // NW
EB = 1280
NBLK = N_EDGES // EB
GPB = EB // 16
DH = HIDDEN // 2


def _sc_segment_sum_t(et, idx):
    mesh = plsc.VectorSubcoreMesh(core_axis_name="c", subcore_axis_name="s")
    cp = pltpu.CompilerParams()
    if "needs_layout_passes" in pltpu.CompilerParams.__dataclass_fields__:
        cp = dataclasses.replace(cp, needs_layout_passes=False)

    @functools.partial(
        pl.kernel,
        mesh=mesh,
        compiler_params=cp,
        out_type=jax.ShapeDtypeStruct((HIDDEN * N_PAD,), jnp.float32),
        scratch_types=[
            pltpu.VMEM((RPW * N_PAD,), jnp.float32),
            pltpu.VMEM((RPW, EB), jnp.float32),
            pltpu.VMEM((RPW, EB), jnp.float32),
            pltpu.VMEM((EB,), jnp.int32),
            pltpu.VMEM((EB,), jnp.int32),
            pltpu.SemaphoreType.DMA,
            pltpu.SemaphoreType.DMA,
        ],
    )
    def k(et_hbm, idx_hbm, out_hbm, acc, e0, e1, i0, i1, sem0, sem1):
        c = lax.axis_index("c")
        s = lax.axis_index("s")
        w = s * NC + c
        r0 = w * RPW

        def gather(j, ebuf, ibuf, sem):
            base = j * EB
            ce = pltpu.make_async_copy(
                et_hbm.at[pl.ds(r0, RPW), pl.ds(base, EB)], ebuf, sem)
            ci = pltpu.make_async_copy(idx_hbm.at[1, pl.ds(base, EB)], ibuf,
                                       sem)
            return ce, ci

        def start(j, ebuf, ibuf, sem):
            ce, ci = gather(j, ebuf, ibuf, sem)
            ce.start()
            ci.start()

        def wait(j, ebuf, ibuf, sem):
            ce, ci = gather(j, ebuf, ibuf, sem)
            ce.wait()
            ci.wait()

        start(0, e0, i0, sem0)

        zv = jnp.zeros((16,), jnp.float32)

        @pl.loop(0, RPW * N_PAD // 16)
        def _(g):
            acc[pl.ds(g * 16, 16)] = zv

        roff = [jnp.full((16,), r * N_PAD, jnp.int32) for r in range(RPW)]

        def accumulate(ebuf, ibuf):
            @plsc.parallel_loop(0, GPB, unroll=8)
            def _(g):
                iv = ibuf[pl.ds(g * 16, 16)]
                for r in range(RPW):
                    vals = ebuf[r, pl.ds(g * 16, 16)]
                    plsc.addupdate_scatter(acc, [iv + roff[r]], vals)

        @pl.loop(0, NBLK - 1, step=2)
        def _(j):
            start(j + 1, e1, i1, sem1)
            wait(j, e0, i0, sem0)
            accumulate(e0, i0)
            start(j + 2, e0, i0, sem0)
            wait(j + 1, e1, i1, sem1)
            accumulate(e1, i1)

        wait(NBLK - 1, e0, i0, sem0)
        accumulate(e0, i0)

        pltpu.sync_copy(acc, out_hbm.at[pl.ds(r0 * N_PAD, RPW * N_PAD)])

    return k(et, idx).reshape(HIDDEN, N_PAD)


TB = 1024


def _transpose_body(in_ref, o_ref):
    o_ref[...] = in_ref[...].T


def _transpose_agg(agg_t):
    return pl.pallas_call(
        _transpose_body,
        grid=(N_PAD // TB,),
        in_specs=[pl.BlockSpec((HIDDEN, TB), lambda i: (0, i))],
        out_specs=pl.BlockSpec((TB, HIDDEN), lambda i: (i, 0)),
        out_shape=jax.ShapeDtypeStruct((N_PAD, HIDDEN), jnp.float32),
        compiler_params=pltpu.CompilerParams(
            dimension_semantics=("parallel",)),
    )(agg_t)


def _mlp_body(a0_ref, a1_ref, v_ref, w1a_ref, w1b_ref, bias1_ref, w2_ref,
              bias2_ref, o_ref):
    dn = (((1,), (1,)), ((), ()))
    h = (
        lax.dot_general(a0_ref[...], w1a_ref[...], dn,
                        preferred_element_type=jnp.float32)
        + lax.dot_general(a1_ref[...], w1b_ref[...], dn,
                          preferred_element_type=jnp.float32)
        + bias1_ref[...]
    )
    sp = jnp.logaddexp(h, 0.0) - SHIFT
    o_ref[...] = (
        lax.dot_general(sp, w2_ref[...], dn,
                        preferred_element_type=jnp.float32)
        + bias2_ref[...]
        + v_ref[...]
    )


def _mlp(agg, v, w1a, w1b, b1, w2, b2):
    rows = 1000
    grid = (N_NODES // rows,)
    return pl.pallas_call(
        _mlp_body,
        grid=grid,
        in_specs=[
            pl.BlockSpec((rows, DH), lambda i: (i, 0)),
            pl.BlockSpec((rows, DH), lambda i: (i, 1)),
            pl.BlockSpec((rows, HIDDEN), lambda i: (i, 0)),
            pl.BlockSpec((HIDDEN, DH), lambda i: (0, 0)),
            pl.BlockSpec((HIDDEN, DH), lambda i: (0, 0)),
            pl.BlockSpec((1, HIDDEN), lambda i: (0, 0)),
            pl.BlockSpec((HIDDEN, HIDDEN), lambda i: (0, 0)),
            pl.BlockSpec((1, HIDDEN), lambda i: (0, 0)),
        ],
        out_specs=pl.BlockSpec((rows, HIDDEN), lambda i: (i, 0)),
        out_shape=jax.ShapeDtypeStruct((N_NODES, HIDDEN), jnp.float32),
        compiler_params=pltpu.CompilerParams(
            dimension_semantics=("parallel",)),
    )(agg, agg, v, w1a, w1b, b1, w2, b2)


def kernel(v, e, edge_index, W1, b1, W2, b2):
    idx = edge_index.astype(jnp.int32)
    et = e.reshape(N_EDGES, HIDDEN).T
    agg_t = _sc_segment_sum_t(et, idx)
    agg = _transpose_agg(agg_t)
    return _mlp(agg, v, W1[:, :DH], W1[:, DH:], b1.reshape(1, HIDDEN), W2,
                b2.reshape(1, HIDDEN))


# speedup vs baseline: 1.0594x; 1.0594x over previous
import dataclasses
import functools

import jax
import jax.numpy as jnp
import numpy as np
from jax import lax
from jax.experimental import pallas as pl
from jax.experimental.pallas import tpu as pltpu
from jax.experimental.pallas import tpu_sc as plsc

HIDDEN = 256
N_NODES = 10000
N_PAD = 10240
N_EDGES = 160000
SHIFT = float(np.log(2.0))

NC = 2
NS = 16
NW = NC * NS
RPW = HIDDEN // NW
EB = 1280
NBLK = N_EDGES // EB
GPB = EB // 16
DH = HIDDEN // 2


def _sc_segment_sum_t(et, idx):
    mesh = plsc.VectorSubcoreMesh(core_axis_name="c", subcore_axis_name="s")
    cp = pltpu.CompilerParams()
    if "needs_layout_passes" in pltpu.CompilerParams.__dataclass_fields__:
        cp = dataclasses.replace(cp, needs_layout_passes=False)

    @functools.partial(
        pl.kernel,
        mesh=mesh,
        compiler_params=cp,
        out_type=jax.ShapeDtypeStruct((HIDDEN, N_PAD), jnp.float32),
        scratch_types=[
            pltpu.VMEM((RPW, N_PAD), jnp.float32),
            pltpu.VMEM((RPW, EB), jnp.float32),
            pltpu.VMEM((RPW, EB), jnp.float32),
            pltpu.VMEM((EB,), jnp.int32),
            pltpu.VMEM((EB,), jnp.int32),
            pltpu.SemaphoreType.DMA,
            pltpu.SemaphoreType.DMA,
        ],
    )
    def k(et_hbm, idx_hbm, out_hbm, acc, e0, e1, i0, i1, sem0, sem1):
        c = lax.axis_index("c")
        s = lax.axis_index("s")
        w = s * NC + c
        r0 = w * RPW

        def gather(j, ebuf, ibuf, sem):
            base = j * EB
            ce = pltpu.make_async_copy(
                et_hbm.at[pl.ds(r0, RPW), pl.ds(base, EB)], ebuf, sem)
            ci = pltpu.make_async_copy(idx_hbm.at[1, pl.ds(base, EB)], ibuf,
                                       sem)
            return ce, ci

        def start(j, ebuf, ibuf, sem):
            ce, ci = gather(j, ebuf, ibuf, sem)
            ce.start()
            ci.start()

        def wait(j, ebuf, ibuf, sem):
            ce, ci = gather(j, ebuf, ibuf, sem)
            ce.wait()
            ci.wait()

        start(0, e0, i0, sem0)

        zv = jnp.zeros((16,), jnp.float32)

        @pl.loop(0, RPW)
        def _(r):
            @pl.loop(0, N_PAD // 16)
            def _(g):
                acc[r, pl.ds(g * 16, 16)] = zv

        def accumulate(ebuf, ibuf):
            @plsc.parallel_loop(0, GPB, unroll=4)
            def _(g):
                iv = ibuf[pl.ds(g * 16, 16)]
                for r in range(RPW):
                    rv = jnp.full((16,), r, jnp.int32)
                    vals = ebuf[r, pl.ds(g * 16, 16)]
                    plsc.addupdate_scatter(acc, [rv, iv], vals)

        @pl.loop(0, NBLK - 1, step=2)
        def _(j):
            start(j + 1, e1, i1, sem1)
            wait(j, e0, i0, sem0)
            accumulate(e0, i0)
            start(j + 2, e0, i0, sem0)
            wait(j + 1, e1, i1, sem1)
            accumulate(e1, i1)

        wait(NBLK - 1, e0, i0, sem0)
        accumulate(e0, i0)

        pltpu.sync_copy(acc, out_hbm.at[pl.ds(r0, RPW)])

    return k(et, idx)


TB = 1024


def _transpose_body(in_ref, o_ref):
    o_ref[...] = in_ref[...].T


def _transpose_agg(agg_t):
    return pl.pallas_call(
        _transpose_body,
        grid=(N_PAD // TB,),
        in_specs=[pl.BlockSpec((HIDDEN, TB), lambda i: (0, i))],
        out_specs=pl.BlockSpec((TB, HIDDEN), lambda i: (i, 0)),
        out_shape=jax.ShapeDtypeStruct((N_PAD, HIDDEN), jnp.float32),
        compiler_params=pltpu.CompilerParams(
            dimension_semantics=("parallel",)),
    )(agg_t)


def _mlp_body(a0_ref, a1_ref, v_ref, w1a_ref, w1b_ref, bias1_ref, w2_ref,
              bias2_ref, o_ref):
    dn = (((1,), (1,)), ((), ()))
    h = (
        lax.dot_general(a0_ref[...], w1a_ref[...], dn,
                        preferred_element_type=jnp.float32)
        + lax.dot_general(a1_ref[...], w1b_ref[...], dn,
                          preferred_element_type=jnp.float32)
        + bias1_ref[...]
    )
    sp = jnp.logaddexp(h, 0.0) - SHIFT
    o_ref[...] = (
        lax.dot_general(sp, w2_ref[...], dn,
                        preferred_element_type=jnp.float32)
        + bias2_ref[...]
        + v_ref[...]
    )


def _mlp(agg, v, w1a, w1b, b1, w2, b2):
    rows = 1000
    grid = (N_NODES // rows,)
    return pl.pallas_call(
        _mlp_body,
        grid=grid,
        in_specs=[
            pl.BlockSpec((rows, DH), lambda i: (i, 0)),
            pl.BlockSpec((rows, DH), lambda i: (i, 1)),
            pl.BlockSpec((rows, HIDDEN), lambda i: (i, 0)),
            pl.BlockSpec((HIDDEN, DH), lambda i: (0, 0)),
            pl.BlockSpec((HIDDEN, DH), lambda i: (0, 0)),
            pl.BlockSpec((1, HIDDEN), lambda i: (0, 0)),
            pl.BlockSpec((HIDDEN, HIDDEN), lambda i: (0, 0)),
            pl.BlockSpec((1, HIDDEN), lambda i: (0, 0)),
        ],
        out_specs=pl.BlockSpec((rows, HIDDEN), lambda i: (i, 0)),
        out_shape=jax.ShapeDtypeStruct((N_NODES, HIDDEN), jnp.float32),
        compiler_params=pltpu.CompilerParams(
            dimension_semantics=("parallel",)),
    )(agg, agg, v, w1a, w1b, b1, w2, b2)


def kernel(v, e, edge_index, W1, b1, W2, b2):
    idx = edge_index.astype(jnp.int32)
    et = e.reshape(N_EDGES, HIDDEN).T
    agg_t = _sc_segment_sum_t(et, idx)
    agg = _transpose_agg(agg_t)
    return _mlp(agg, v, W1[:, :DH], W1[:, DH:], b1.reshape(1, HIDDEN), W2,
                b2.reshape(1, HIDDEN))
